# baseline (speedup 1.0000x reference)
import functools

import jax
import jax.numpy as jnp
from jax import lax
from jax.experimental import pallas as pl
from jax.experimental.pallas import tpu as pltpu
from jax.experimental.pallas import tpu_sc as plsc

_N, _E, _D, _H, _OUT, _G = 10000, 320000, 128, 128, 12, 256
_HD = _D // 2
_NC, _NS = 2, 16
_CH = 128
_NCH = 160
_EPT = _NCH * _CH
_EPAD = _NS * _EPT
_NBUF = 8
_NP = 10240
_RPT = _NP // _NS

_sc_mesh = plsc.VectorSubcoreMesh(
    core_axis_name="c", subcore_axis_name="s", num_cores=_NC, num_subcores=_NS)


@functools.partial(
    pl.kernel,
    out_type=jax.ShapeDtypeStruct((_NC, _NP, _HD), jnp.float32),
    mesh=_sc_mesh,
    scratch_types=[
        pltpu.VMEM_SHARED((_NP, _HD), jnp.float32),
        [pltpu.VMEM((2, _CH), jnp.int32)] * _NBUF,
        [pltpu.VMEM((_CH, _HD), jnp.float32)] * _NBUF,
        [pltpu.SemaphoreType.DMA] * _NBUF,
        [pltpu.SemaphoreType.DMA] * _NBUF,
        [pltpu.SemaphoreType.DMA] * _NBUF,
    ],
    compiler_params=pltpu.CompilerParams(use_tc_tiling_on_sc=False),
)
def _sc_agg(x0_hbm, x1_hbm, e_hbm, z_hbm, out_hbm,
            acc_sh, ibufs, rows, isems, gsems, ssems):
    c = lax.axis_index("c")
    s = lax.axis_index("s")
    pltpu.sync_copy(z_hbm, rows[0])
    for k in range(_RPT // _CH):
        pltpu.sync_copy(rows[0], acc_sh.at[pl.ds(s * _RPT + k * _CH, _CH)])
    plsc.subcore_barrier()

    def edge_loop(x_hbm):
        for k in range(4):
            pltpu.async_copy(e_hbm.at[s, k], ibufs[k], isems[k])
        for k in range(2):
            pltpu.make_async_copy(e_hbm.at[s, k], ibufs[k], isems[k]).wait()
            pltpu.async_copy(x_hbm.at[ibufs[k].at[0]], rows[k], gsems[k])

        @pl.loop(0, _NCH, step=_NBUF)
        def _(i0):
            for b in range(_NBUF):
                i = i0 + b
                bs = (b + 4) % _NBUF
                bg = (b + 2) % _NBUF

                @pl.when(i >= 4)
                def _():
                    pltpu.make_async_copy(rows[bs],
                                          acc_sh.at[ibufs[bs].at[1]],
                                          ssems[bs]).wait()

                @pl.when(i + 4 < _NCH)
                def _():
                    pltpu.async_copy(e_hbm.at[s, i + 4], ibufs[bs], isems[bs])

                @pl.when(i + 2 < _NCH)
                def _():
                    pltpu.make_async_copy(e_hbm.at[s, i + 2], ibufs[bg],
                                          isems[bg]).wait()
                    pltpu.async_copy(x_hbm.at[ibufs[bg].at[0]], rows[bg],
                                     gsems[bg])

                pltpu.make_async_copy(x_hbm.at[ibufs[b].at[0]], rows[b],
                                      gsems[b]).wait()
                pltpu.async_copy(rows[b], acc_sh.at[ibufs[b].at[1]], ssems[b],
                                 add=True)

        for b in range(4, _NBUF):
            pltpu.make_async_copy(rows[b], acc_sh.at[ibufs[b].at[1]],
                                  ssems[b]).wait()

    @pl.when(c == 0)
    def _():
        edge_loop(x0_hbm)

    @pl.when(c == 1)
    def _():
        edge_loop(x1_hbm)

    plsc.subcore_barrier()
    pltpu.sync_copy(acc_sh.at[pl.ds(s * _RPT, _RPT)],
                    out_hbm.at[c, pl.ds(s * _RPT, _RPT)])


def _mlp_bn(a, b, w1_ref, b1_ref, w2_ref, b2_ref, g_ref, bt_ref):
    h = jnp.dot(a, w1_ref[:_HD], preferred_element_type=jnp.float32)
    h += jnp.dot(b, w1_ref[_HD:], preferred_element_type=jnp.float32)
    h = jnp.maximum(h + b1_ref[...], 0.0)
    h = jnp.dot(h, w2_ref[...], preferred_element_type=jnp.float32) + b2_ref[...]
    mu = jnp.mean(h, axis=0, keepdims=True)
    var = jnp.mean(jnp.square(h - mu), axis=0, keepdims=True)
    h = (h - mu) * lax.rsqrt(var + 1e-5) * g_ref[...] + bt_ref[...]
    return jnp.maximum(h, 0.0)


def _dense_body(xl_ref, xh_ref, p_ref, w1_ref, b1_ref, w2_ref, b2_ref,
                g_ref, bt_ref, ol_ref, oh_ref):
    a = xl_ref[...] + p_ref[0, :_N]
    b = xh_ref[...] + p_ref[1, :_N]
    h = _mlp_bn(a, b, w1_ref, b1_ref, w2_ref, b2_ref, g_ref, bt_ref)
    ol_ref[...] = h[:, :_HD]
    oh_ref[...] = h[:, _HD:]


_dense = pl.pallas_call(
    _dense_body,
    out_shape=[jax.ShapeDtypeStruct((_N, _HD), jnp.float32),
               jax.ShapeDtypeStruct((_N, _HD), jnp.float32)],
)


def _final_body(xl_ref, xh_ref, p_ref, batch_ref, w1_ref, b1_ref, w2_ref,
                b2_ref, g_ref, bt_ref, wh1_ref, bh1_ref, wh2_ref, bh2_ref,
                o_ref):
    a = xl_ref[...] + p_ref[0, :_N]
    b = xh_ref[...] + p_ref[1, :_N]
    h = _mlp_bn(a, b, w1_ref, b1_ref, w2_ref, b2_ref, g_ref, bt_ref)
    gids = lax.broadcasted_iota(jnp.int32, (_G, _N), 0)
    onehot = (batch_ref[...] == gids).astype(jnp.float32)
    pool = jnp.dot(onehot, h, preferred_element_type=jnp.float32)
    q = jnp.maximum(
        jnp.dot(pool, wh1_ref[...], preferred_element_type=jnp.float32)
        + bh1_ref[...], 0.0)
    o_ref[...] = jnp.dot(q, wh2_ref[...],
                         preferred_element_type=jnp.float32) + bh2_ref[...]


_final = pl.pallas_call(
    _final_body,
    out_shape=jax.ShapeDtypeStruct((_G, _OUT), jnp.float32),
)


def kernel(x, edge_index, batch, W1_0, b1_0, W2_0, b2_0, g_0, bt_0,
           W1_1, b1_1, W2_1, b2_1, g_1, bt_1,
           W1_2, b1_2, W2_2, b2_2, g_2, bt_2, Wh1, bh1, Wh2, bh2):
    pad = _EPAD - _E
    src = jnp.concatenate([edge_index[0], jnp.zeros((pad,), jnp.int32)])
    pdst = _N + jnp.arange(pad, dtype=jnp.int32) % (_NP - _N)
    dst = jnp.concatenate([edge_index[1], pdst])
    e = jnp.stack([src.reshape(_NS, _NCH, _CH),
                   dst.reshape(_NS, _NCH, _CH)], axis=2)
    zrows = jnp.zeros((_CH, _HD), jnp.float32)
    r2 = lambda v: v.reshape(1, -1)
    hl, hh = x[:, :_HD], x[:, _HD:]

    p = _sc_agg(hl, hh, e, zrows)
    hl, hh = _dense(hl, hh, p, W1_0, r2(b1_0), W2_0, r2(b2_0), r2(g_0),
                    r2(bt_0))
    p = _sc_agg(hl, hh, e, zrows)
    hl, hh = _dense(hl, hh, p, W1_1, r2(b1_1), W2_1, r2(b2_1), r2(g_1),
                    r2(bt_1))
    p = _sc_agg(hl, hh, e, zrows)
    return _final(hl, hh, p, batch.reshape(1, -1), W1_2, r2(b1_2), W2_2,
                  r2(b2_2), r2(g_2), r2(bt_2), Wh1, r2(bh1), Wh2, r2(bh2))


# speedup vs baseline: 2.1937x; 2.1937x over previous
import functools

import jax
import jax.numpy as jnp
from jax import lax
from jax.experimental import pallas as pl
from jax.experimental.pallas import tpu as pltpu
from jax.experimental.pallas import tpu_sc as plsc

_N, _E, _D, _H, _OUT, _G = 10000, 320000, 128, 128, 12, 256
_HD = _D // 2
_NC, _NS = 2, 16
_CH = 128
_NCH = 160
_EPT = _NCH * _CH
_EPAD = _NS * _EPT
_NBUF = 4
_NIB = 8
_NP = 10240
_RPT = _NP // _NS

_sc_mesh = plsc.VectorSubcoreMesh(
    core_axis_name="c", subcore_axis_name="s", num_cores=_NC, num_subcores=_NS)


@functools.partial(
    pl.kernel,
    out_type=jax.ShapeDtypeStruct((_NC, _NP, _HD), jnp.float32),
    mesh=_sc_mesh,
    scratch_types=[
        pltpu.VMEM_SHARED((_NP, _HD), jnp.float32),
        pltpu.VMEM_SHARED((_NP, _HD), jnp.float32),
        [pltpu.VMEM((2, _CH), jnp.int32)] * _NIB,
        [pltpu.VMEM((_CH, _HD), jnp.float32)] * _NBUF,
        [pltpu.SemaphoreType.DMA] * _NIB,
        [pltpu.SemaphoreType.DMA] * _NBUF,
        [pltpu.SemaphoreType.DMA] * _NBUF,
    ],
    compiler_params=pltpu.CompilerParams(use_tc_tiling_on_sc=False),
)
def _sc_agg(x0_hbm, x1_hbm, e_hbm, z_hbm, out_hbm,
            acc_sh, x_sh, ibufs, rows, isems, gsems, ssems):
    c = lax.axis_index("c")
    s = lax.axis_index("s")
    pltpu.sync_copy(z_hbm, rows[0])
    for k in range(_RPT // _CH):
        pltpu.sync_copy(rows[0], acc_sh.at[pl.ds(s * _RPT + k * _CH, _CH)])

    @pl.when(c == 0)
    def _():
        pltpu.sync_copy(x0_hbm.at[pl.ds(s * _RPT, _RPT)],
                        x_sh.at[pl.ds(s * _RPT, _RPT)])

    @pl.when(c == 1)
    def _():
        pltpu.sync_copy(x1_hbm.at[pl.ds(s * _RPT, _RPT)],
                        x_sh.at[pl.ds(s * _RPT, _RPT)])

    plsc.subcore_barrier()

    for k in range(4):
        pltpu.async_copy(e_hbm.at[s, k], ibufs[k], isems[k])
    for k in range(2):
        pltpu.make_async_copy(e_hbm.at[s, k], ibufs[k], isems[k]).wait()
        pltpu.async_copy(x_sh.at[ibufs[k].at[0]], rows[k], gsems[k])

    @pl.loop(0, _NCH, step=_NIB)
    def _(i0):
        for b in range(_NIB):
            i = i0 + b
            br = b % _NBUF
            bg = (b + 2) % _NBUF
            bi = (b + 4) % _NIB
            bgi = (b + 2) % _NIB

            @pl.when(i >= 2)
            def _():
                pltpu.make_async_copy(rows[bg],
                                      acc_sh.at[ibufs[(b - 2) % _NIB].at[1]],
                                      ssems[bg]).wait()

            @pl.when(i + 4 < _NCH)
            def _():
                pltpu.async_copy(e_hbm.at[s, i + 4], ibufs[bi], isems[bi])

            @pl.when(i + 2 < _NCH)
            def _():
                pltpu.make_async_copy(e_hbm.at[s, i + 2], ibufs[bgi],
                                      isems[bgi]).wait()
                pltpu.async_copy(x_sh.at[ibufs[bgi].at[0]], rows[bg],
                                 gsems[bg])

            pltpu.make_async_copy(x_sh.at[ibufs[b].at[0]], rows[br],
                                  gsems[br]).wait()
            pltpu.async_copy(rows[br], acc_sh.at[ibufs[b].at[1]], ssems[br],
                             add=True)

    pltpu.make_async_copy(rows[(_NCH - 2) % _NBUF],
                          acc_sh.at[ibufs[(_NCH - 2) % _NIB].at[1]],
                          ssems[(_NCH - 2) % _NBUF]).wait()
    pltpu.make_async_copy(rows[(_NCH - 1) % _NBUF],
                          acc_sh.at[ibufs[(_NCH - 1) % _NIB].at[1]],
                          ssems[(_NCH - 1) % _NBUF]).wait()

    plsc.subcore_barrier()
    pltpu.sync_copy(acc_sh.at[pl.ds(s * _RPT, _RPT)],
                    out_hbm.at[c, pl.ds(s * _RPT, _RPT)])


def _mlp_bn(a, b, w1_ref, b1_ref, w2_ref, b2_ref, g_ref, bt_ref):
    h = jnp.dot(a, w1_ref[:_HD], preferred_element_type=jnp.float32)
    h += jnp.dot(b, w1_ref[_HD:], preferred_element_type=jnp.float32)
    h = jnp.maximum(h + b1_ref[...], 0.0)
    h = jnp.dot(h, w2_ref[...], preferred_element_type=jnp.float32) + b2_ref[...]
    mu = jnp.mean(h, axis=0, keepdims=True)
    var = jnp.mean(jnp.square(h - mu), axis=0, keepdims=True)
    h = (h - mu) * lax.rsqrt(var + 1e-5) * g_ref[...] + bt_ref[...]
    return jnp.maximum(h, 0.0)


def _dense_body(xl_ref, xh_ref, p_ref, w1_ref, b1_ref, w2_ref, b2_ref,
                g_ref, bt_ref, ol_ref, oh_ref):
    a = xl_ref[:_N] + p_ref[0, :_N]
    b = xh_ref[:_N] + p_ref[1, :_N]
    h = _mlp_bn(a, b, w1_ref, b1_ref, w2_ref, b2_ref, g_ref, bt_ref)
    ol_ref[pl.ds(0, _N)] = h[:, :_HD]
    oh_ref[pl.ds(0, _N)] = h[:, _HD:]


_dense = pl.pallas_call(
    _dense_body,
    out_shape=[jax.ShapeDtypeStruct((_NP, _HD), jnp.float32),
               jax.ShapeDtypeStruct((_NP, _HD), jnp.float32)],
)


def _final_body(xl_ref, xh_ref, p_ref, batch_ref, w1_ref, b1_ref, w2_ref,
                b2_ref, g_ref, bt_ref, wh1_ref, bh1_ref, wh2_ref, bh2_ref,
                o_ref):
    a = xl_ref[:_N] + p_ref[0, :_N]
    b = xh_ref[:_N] + p_ref[1, :_N]
    h = _mlp_bn(a, b, w1_ref, b1_ref, w2_ref, b2_ref, g_ref, bt_ref)
    gids = lax.broadcasted_iota(jnp.int32, (_G, _N), 0)
    onehot = (batch_ref[...] == gids).astype(jnp.float32)
    pool = jnp.dot(onehot, h, preferred_element_type=jnp.float32)
    q = jnp.maximum(
        jnp.dot(pool, wh1_ref[...], preferred_element_type=jnp.float32)
        + bh1_ref[...], 0.0)
    o_ref[...] = jnp.dot(q, wh2_ref[...],
                         preferred_element_type=jnp.float32) + bh2_ref[...]


_final = pl.pallas_call(
    _final_body,
    out_shape=jax.ShapeDtypeStruct((_G, _OUT), jnp.float32),
)


def kernel(x, edge_index, batch, W1_0, b1_0, W2_0, b2_0, g_0, bt_0,
           W1_1, b1_1, W2_1, b2_1, g_1, bt_1,
           W1_2, b1_2, W2_2, b2_2, g_2, bt_2, Wh1, bh1, Wh2, bh2):
    pad = _EPAD - _E
    src = jnp.concatenate([edge_index[0], jnp.zeros((pad,), jnp.int32)])
    pdst = _N + jnp.arange(pad, dtype=jnp.int32) % (_NP - _N)
    dst = jnp.concatenate([edge_index[1], pdst])
    e = jnp.stack([src.reshape(_NS, _NCH, _CH),
                   dst.reshape(_NS, _NCH, _CH)], axis=2)
    zrows = jnp.zeros((_CH, _HD), jnp.float32)
    r2 = lambda v: v.reshape(1, -1)
    xp = jnp.concatenate([x, jnp.zeros((_NP - _N, _D), jnp.float32)])
    hl, hh = xp[:, :_HD], xp[:, _HD:]

    p = _sc_agg(hl, hh, e, zrows)
    hl, hh = _dense(hl, hh, p, W1_0, r2(b1_0), W2_0, r2(b2_0), r2(g_0),
                    r2(bt_0))
    p = _sc_agg(hl, hh, e, zrows)
    hl, hh = _dense(hl, hh, p, W1_1, r2(b1_1), W2_1, r2(b2_1), r2(g_1),
                    r2(bt_1))
    p = _sc_agg(hl, hh, e, zrows)
    return _final(hl, hh, p, batch.reshape(1, -1), W1_2, r2(b1_2), W2_2,
                  r2(b2_2), r2(g_2), r2(bt_2), Wh1, r2(bh1), Wh2, r2(bh2))


# speedup vs baseline: 2.2517x; 1.0264x over previous
import functools

import jax
import jax.numpy as jnp
from jax import lax
from jax.experimental import pallas as pl
from jax.experimental.pallas import tpu as pltpu
from jax.experimental.pallas import tpu_sc as plsc

_N, _E, _D, _H, _OUT, _G = 10000, 320000, 128, 128, 12, 256
_HD = _D // 2
_NC, _NS = 2, 16
_CH = 128
_NCH = 160
_EPT = _NCH * _CH
_EPAD = _NS * _EPT
_NBUF = 4
_NIB = 8
_NP = 10240
_RPT = _NP // _NS

_sc_mesh = plsc.VectorSubcoreMesh(
    core_axis_name="c", subcore_axis_name="s", num_cores=_NC, num_subcores=_NS)


@functools.partial(
    pl.kernel,
    out_type=jax.ShapeDtypeStruct((_NC, _NP, _HD), jnp.float32),
    mesh=_sc_mesh,
    scratch_types=[
        pltpu.VMEM_SHARED((_NP, _HD), jnp.float32),
        pltpu.VMEM_SHARED((_NP, _HD), jnp.float32),
        [pltpu.VMEM((2, _CH), jnp.int32)] * _NIB,
        [pltpu.VMEM((_CH, _HD), jnp.float32)] * _NBUF,
        [pltpu.SemaphoreType.DMA] * _NIB,
        [pltpu.SemaphoreType.DMA] * _NBUF,
        [pltpu.SemaphoreType.DMA] * _NBUF,
    ],
    compiler_params=pltpu.CompilerParams(use_tc_tiling_on_sc=False),
)
def _sc_agg(x0_hbm, x1_hbm, e_hbm, out_hbm,
            acc_sh, x_sh, ibufs, rows, isems, gsems, ssems):
    c = lax.axis_index("c")
    s = lax.axis_index("s")
    @pl.when(c == 0)
    def _():
        pltpu.sync_copy(x0_hbm.at[pl.ds(s * _RPT, _RPT)],
                        x_sh.at[pl.ds(s * _RPT, _RPT)])
        pltpu.sync_copy(x0_hbm.at[pl.ds(s * _RPT, _RPT)],
                        acc_sh.at[pl.ds(s * _RPT, _RPT)])

    @pl.when(c == 1)
    def _():
        pltpu.sync_copy(x1_hbm.at[pl.ds(s * _RPT, _RPT)],
                        x_sh.at[pl.ds(s * _RPT, _RPT)])
        pltpu.sync_copy(x1_hbm.at[pl.ds(s * _RPT, _RPT)],
                        acc_sh.at[pl.ds(s * _RPT, _RPT)])

    plsc.subcore_barrier()

    for k in range(4):
        pltpu.async_copy(e_hbm.at[s, k], ibufs[k], isems[k])
    for k in range(2):
        pltpu.make_async_copy(e_hbm.at[s, k], ibufs[k], isems[k]).wait()
        pltpu.async_copy(x_sh.at[ibufs[k].at[0]], rows[k], gsems[k])

    @pl.loop(0, _NCH, step=_NIB)
    def _(i0):
        for b in range(_NIB):
            i = i0 + b
            br = b % _NBUF
            bg = (b + 2) % _NBUF
            bi = (b + 4) % _NIB
            bgi = (b + 2) % _NIB

            @pl.when(i >= 2)
            def _():
                pltpu.make_async_copy(rows[bg],
                                      acc_sh.at[ibufs[(b - 2) % _NIB].at[1]],
                                      ssems[bg]).wait()

            @pl.when(i + 4 < _NCH)
            def _():
                pltpu.async_copy(e_hbm.at[s, i + 4], ibufs[bi], isems[bi])

            @pl.when(i + 2 < _NCH)
            def _():
                pltpu.make_async_copy(e_hbm.at[s, i + 2], ibufs[bgi],
                                      isems[bgi]).wait()
                pltpu.async_copy(x_sh.at[ibufs[bgi].at[0]], rows[bg],
                                 gsems[bg])

            pltpu.make_async_copy(x_sh.at[ibufs[b].at[0]], rows[br],
                                  gsems[br]).wait()
            pltpu.async_copy(rows[br], acc_sh.at[ibufs[b].at[1]], ssems[br],
                             add=True)

    pltpu.make_async_copy(rows[(_NCH - 2) % _NBUF],
                          acc_sh.at[ibufs[(_NCH - 2) % _NIB].at[1]],
                          ssems[(_NCH - 2) % _NBUF]).wait()
    pltpu.make_async_copy(rows[(_NCH - 1) % _NBUF],
                          acc_sh.at[ibufs[(_NCH - 1) % _NIB].at[1]],
                          ssems[(_NCH - 1) % _NBUF]).wait()

    plsc.subcore_barrier()
    pltpu.sync_copy(acc_sh.at[pl.ds(s * _RPT, _RPT)],
                    out_hbm.at[c, pl.ds(s * _RPT, _RPT)])


def _mlp_bn(a, b, w1_ref, b1_ref, w2_ref, b2_ref, g_ref, bt_ref):
    h = jnp.dot(a, w1_ref[:_HD], preferred_element_type=jnp.float32)
    h += jnp.dot(b, w1_ref[_HD:], preferred_element_type=jnp.float32)
    h = jnp.maximum(h + b1_ref[...], 0.0)
    h = jnp.dot(h, w2_ref[...], preferred_element_type=jnp.float32) + b2_ref[...]
    mu = jnp.mean(h, axis=0, keepdims=True)
    var = jnp.mean(jnp.square(h - mu), axis=0, keepdims=True)
    h = (h - mu) * lax.rsqrt(var + 1e-5) * g_ref[...] + bt_ref[...]
    return jnp.maximum(h, 0.0)


def _dense_body(p_ref, w1_ref, b1_ref, w2_ref, b2_ref,
                g_ref, bt_ref, ol_ref, oh_ref):
    h = _mlp_bn(p_ref[0, :_N], p_ref[1, :_N],
                w1_ref, b1_ref, w2_ref, b2_ref, g_ref, bt_ref)
    ol_ref[pl.ds(0, _N)] = h[:, :_HD]
    oh_ref[pl.ds(0, _N)] = h[:, _HD:]


_dense = pl.pallas_call(
    _dense_body,
    out_shape=[jax.ShapeDtypeStruct((_NP, _HD), jnp.float32),
               jax.ShapeDtypeStruct((_NP, _HD), jnp.float32)],
)


def _final_body(p_ref, batch_ref, w1_ref, b1_ref, w2_ref,
                b2_ref, g_ref, bt_ref, wh1_ref, bh1_ref, wh2_ref, bh2_ref,
                o_ref):
    h = _mlp_bn(p_ref[0, :_N], p_ref[1, :_N],
                w1_ref, b1_ref, w2_ref, b2_ref, g_ref, bt_ref)
    gids = lax.broadcasted_iota(jnp.int32, (_G, _N), 0)
    onehot = (batch_ref[...] == gids).astype(jnp.float32)
    pool = jnp.dot(onehot, h, preferred_element_type=jnp.float32)
    q = jnp.maximum(
        jnp.dot(pool, wh1_ref[...], preferred_element_type=jnp.float32)
        + bh1_ref[...], 0.0)
    o_ref[...] = jnp.dot(q, wh2_ref[...],
                         preferred_element_type=jnp.float32) + bh2_ref[...]


_final = pl.pallas_call(
    _final_body,
    out_shape=jax.ShapeDtypeStruct((_G, _OUT), jnp.float32),
)


def kernel(x, edge_index, batch, W1_0, b1_0, W2_0, b2_0, g_0, bt_0,
           W1_1, b1_1, W2_1, b2_1, g_1, bt_1,
           W1_2, b1_2, W2_2, b2_2, g_2, bt_2, Wh1, bh1, Wh2, bh2):
    pad = _EPAD - _E
    src = jnp.concatenate([edge_index[0], jnp.zeros((pad,), jnp.int32)])
    pdst = _N + jnp.arange(pad, dtype=jnp.int32) % (_NP - _N)
    dst = jnp.concatenate([edge_index[1], pdst])
    e = jnp.stack([src.reshape(_NS, _NCH, _CH),
                   dst.reshape(_NS, _NCH, _CH)], axis=2)
    r2 = lambda v: v.reshape(1, -1)
    xp = jnp.concatenate([x, jnp.zeros((_NP - _N, _D), jnp.float32)])
    hl, hh = xp[:, :_HD], xp[:, _HD:]

    p = _sc_agg(hl, hh, e)
    hl, hh = _dense(p, W1_0, r2(b1_0), W2_0, r2(b2_0), r2(g_0), r2(bt_0))
    p = _sc_agg(hl, hh, e)
    hl, hh = _dense(p, W1_1, r2(b1_1), W2_1, r2(b2_1), r2(g_1), r2(bt_1))
    p = _sc_agg(hl, hh, e)
    return _final(p, batch.reshape(1, -1), W1_2, r2(b1_2), W2_2,
                  r2(b2_2), r2(g_2), r2(bt_2), Wh1, r2(bh1), Wh2, r2(bh2))


# speedup vs baseline: 2.3769x; 1.0556x over previous
import functools

import jax
import jax.numpy as jnp
from jax import lax
from jax.experimental import pallas as pl
from jax.experimental.pallas import tpu as pltpu
from jax.experimental.pallas import tpu_sc as plsc

_N, _E, _D, _H, _OUT, _G = 10000, 320000, 128, 128, 12, 256
_HD = _D // 2
_NC, _NS = 2, 16
_CH = 128
_NCH = 160
_EPT = _NCH * _CH
_EPAD = _NS * _EPT
_NBUF = 4
_NIB = 8
_NP = 10240
_RPT = _NP // _NS

_sc_mesh = plsc.VectorSubcoreMesh(
    core_axis_name="c", subcore_axis_name="s", num_cores=_NC, num_subcores=_NS)


@functools.partial(
    pl.kernel,
    out_type=jax.ShapeDtypeStruct((_NC, _NP, _HD), jnp.float32),
    mesh=_sc_mesh,
    scratch_types=[
        pltpu.VMEM_SHARED((_NP, _HD), jnp.float32),
        pltpu.VMEM_SHARED((_NP, _HD), jnp.float32),
        [pltpu.VMEM((2, _CH), jnp.int32)] * _NIB,
        [pltpu.VMEM((_CH, _HD), jnp.float32)] * _NBUF,
        [pltpu.SemaphoreType.DMA] * _NIB,
        [pltpu.SemaphoreType.DMA] * _NBUF,
        [pltpu.SemaphoreType.DMA] * _NBUF,
    ],
    compiler_params=pltpu.CompilerParams(use_tc_tiling_on_sc=False),
)
def _sc_agg(xf_hbm, e_hbm, out_hbm,
            acc_sh, x_sh, ibufs, rows, isems, gsems, ssems):
    c = lax.axis_index("c")
    s = lax.axis_index("s")
    @pl.when(c == 0)
    def _():
        pltpu.sync_copy(xf_hbm.at[pl.ds(s * _RPT, _RPT), pl.ds(0, _HD)],
                        x_sh.at[pl.ds(s * _RPT, _RPT)])
        pltpu.sync_copy(xf_hbm.at[pl.ds(s * _RPT, _RPT), pl.ds(0, _HD)],
                        acc_sh.at[pl.ds(s * _RPT, _RPT)])

    @pl.when(c == 1)
    def _():
        pltpu.sync_copy(xf_hbm.at[pl.ds(s * _RPT, _RPT), pl.ds(_HD, _HD)],
                        x_sh.at[pl.ds(s * _RPT, _RPT)])
        pltpu.sync_copy(xf_hbm.at[pl.ds(s * _RPT, _RPT), pl.ds(_HD, _HD)],
                        acc_sh.at[pl.ds(s * _RPT, _RPT)])

    plsc.subcore_barrier()

    def fetch_idx(i, k):
        pltpu.async_copy(e_hbm.at[0, s, i], ibufs[k].at[0], isems[k])
        pltpu.async_copy(e_hbm.at[1, s, i], ibufs[k].at[1], isems[k])

    def wait_idx(i, k):
        pltpu.make_async_copy(e_hbm.at[0, s, i], ibufs[k].at[0],
                              isems[k]).wait()
        pltpu.make_async_copy(e_hbm.at[1, s, i], ibufs[k].at[1],
                              isems[k]).wait()

    for k in range(4):
        fetch_idx(k, k)
    for k in range(2):
        wait_idx(k, k)
        pltpu.async_copy(x_sh.at[ibufs[k].at[0]], rows[k], gsems[k])

    @pl.loop(0, _NCH, step=_NIB)
    def _(i0):
        for b in range(_NIB):
            i = i0 + b
            br = b % _NBUF
            bg = (b + 2) % _NBUF
            bi = (b + 4) % _NIB
            bgi = (b + 2) % _NIB

            @pl.when(i >= 2)
            def _():
                pltpu.make_async_copy(rows[bg],
                                      acc_sh.at[ibufs[(b - 2) % _NIB].at[1]],
                                      ssems[bg]).wait()

            @pl.when(i + 4 < _NCH)
            def _():
                fetch_idx(i + 4, bi)

            @pl.when(i + 2 < _NCH)
            def _():
                wait_idx(i + 2, bgi)
                pltpu.async_copy(x_sh.at[ibufs[bgi].at[0]], rows[bg],
                                 gsems[bg])

            pltpu.make_async_copy(x_sh.at[ibufs[b].at[0]], rows[br],
                                  gsems[br]).wait()
            pltpu.async_copy(rows[br], acc_sh.at[ibufs[b].at[1]], ssems[br],
                             add=True)

    pltpu.make_async_copy(rows[(_NCH - 2) % _NBUF],
                          acc_sh.at[ibufs[(_NCH - 2) % _NIB].at[1]],
                          ssems[(_NCH - 2) % _NBUF]).wait()
    pltpu.make_async_copy(rows[(_NCH - 1) % _NBUF],
                          acc_sh.at[ibufs[(_NCH - 1) % _NIB].at[1]],
                          ssems[(_NCH - 1) % _NBUF]).wait()

    plsc.subcore_barrier()
    pltpu.sync_copy(acc_sh.at[pl.ds(s * _RPT, _RPT)],
                    out_hbm.at[c, pl.ds(s * _RPT, _RPT)])


def _mlp_bn(a, b, w1_ref, b1_ref, w2_ref, b2_ref, g_ref, bt_ref):
    h = jnp.dot(a, w1_ref[:_HD], preferred_element_type=jnp.float32)
    h += jnp.dot(b, w1_ref[_HD:], preferred_element_type=jnp.float32)
    h = jnp.maximum(h + b1_ref[...], 0.0)
    h = jnp.dot(h, w2_ref[...], preferred_element_type=jnp.float32) + b2_ref[...]
    mu = jnp.mean(h, axis=0, keepdims=True)
    var = jnp.mean(jnp.square(h - mu), axis=0, keepdims=True)
    h = (h - mu) * lax.rsqrt(var + 1e-5) * g_ref[...] + bt_ref[...]
    return jnp.maximum(h, 0.0)


def _dense_body(p_ref, w1_ref, b1_ref, w2_ref, b2_ref,
                g_ref, bt_ref, o_ref):
    h = _mlp_bn(p_ref[0, :_N], p_ref[1, :_N],
                w1_ref, b1_ref, w2_ref, b2_ref, g_ref, bt_ref)
    o_ref[pl.ds(0, _N)] = h


_dense = pl.pallas_call(
    _dense_body,
    out_shape=jax.ShapeDtypeStruct((_NP, _D), jnp.float32),
)


def _final_body(p_ref, batch_ref, w1_ref, b1_ref, w2_ref,
                b2_ref, g_ref, bt_ref, wh1_ref, bh1_ref, wh2_ref, bh2_ref,
                o_ref):
    h = _mlp_bn(p_ref[0, :_N], p_ref[1, :_N],
                w1_ref, b1_ref, w2_ref, b2_ref, g_ref, bt_ref)
    gids = lax.broadcasted_iota(jnp.int32, (_G, _N), 0)
    onehot = (batch_ref[...] == gids).astype(jnp.float32)
    pool = jnp.dot(onehot, h, preferred_element_type=jnp.float32)
    q = jnp.maximum(
        jnp.dot(pool, wh1_ref[...], preferred_element_type=jnp.float32)
        + bh1_ref[...], 0.0)
    o_ref[...] = jnp.dot(q, wh2_ref[...],
                         preferred_element_type=jnp.float32) + bh2_ref[...]


_final = pl.pallas_call(
    _final_body,
    out_shape=jax.ShapeDtypeStruct((_G, _OUT), jnp.float32),
)


def kernel(x, edge_index, batch, W1_0, b1_0, W2_0, b2_0, g_0, bt_0,
           W1_1, b1_1, W2_1, b2_1, g_1, bt_1,
           W1_2, b1_2, W2_2, b2_2, g_2, bt_2, Wh1, bh1, Wh2, bh2):
    e = jnp.pad(edge_index, ((0, 0), (0, _EPAD - _E)),
                constant_values=_NP - 1).reshape(2, _NS, _NCH, _CH)
    r2 = lambda v: v.reshape(1, -1)
    xp = jnp.concatenate([x, jnp.zeros((_NP - _N, _D), jnp.float32)])

    p = _sc_agg(xp, e)
    h = _dense(p, W1_0, r2(b1_0), W2_0, r2(b2_0), r2(g_0), r2(bt_0))
    p = _sc_agg(h, e)
    h = _dense(p, W1_1, r2(b1_1), W2_1, r2(b2_1), r2(g_1), r2(bt_1))
    p = _sc_agg(h, e)
    return _final(p, batch.reshape(1, -1), W1_2, r2(b1_2), W2_2,
                  r2(b2_2), r2(g_2), r2(bt_2), Wh1, r2(bh1), Wh2, r2(bh2))


# speedup vs baseline: 3.2667x; 1.3744x over previous
import functools

import jax
import jax.numpy as jnp
from jax import lax
from jax.experimental import pallas as pl
from jax.experimental.pallas import tpu as pltpu
from jax.experimental.pallas import tpu_sc as plsc

_N, _E, _D, _H, _OUT, _G = 10000, 320000, 128, 128, 12, 256
_HD = _D // 2
_NC, _NS = 2, 16
_CH = 128
_NCH = 160
_EPT = _NCH * _CH
_EPAD = _NS * _EPT
_NBUF = 4
_NIB = 8
_NP = 10240
_RPT = _NP // _NS

_sc_mesh = plsc.VectorSubcoreMesh(
    core_axis_name="c", subcore_axis_name="s", num_cores=_NC, num_subcores=_NS)


@functools.partial(
    pl.kernel,
    out_type=jax.ShapeDtypeStruct((_NC, _NP, _HD), jnp.bfloat16),
    mesh=_sc_mesh,
    scratch_types=[
        pltpu.VMEM_SHARED((_NP, _HD), jnp.bfloat16),
        pltpu.VMEM_SHARED((_NP, _HD), jnp.bfloat16),
        [pltpu.VMEM((2, _CH), jnp.int32)] * _NIB,
        [pltpu.VMEM((_CH, _HD), jnp.bfloat16)] * _NBUF,
        [pltpu.SemaphoreType.DMA] * _NIB,
        [pltpu.SemaphoreType.DMA] * _NBUF,
        [pltpu.SemaphoreType.DMA] * _NBUF,
    ],
    compiler_params=pltpu.CompilerParams(use_tc_tiling_on_sc=False),
)
def _sc_agg(xf_hbm, e_hbm, out_hbm,
            acc_sh, x_sh, ibufs, rows, isems, gsems, ssems):
    c = lax.axis_index("c")
    s = lax.axis_index("s")
    @pl.when(c == 0)
    def _():
        pltpu.sync_copy(xf_hbm.at[pl.ds(s * _RPT, _RPT), pl.ds(0, _HD)],
                        x_sh.at[pl.ds(s * _RPT, _RPT)])
        pltpu.sync_copy(xf_hbm.at[pl.ds(s * _RPT, _RPT), pl.ds(0, _HD)],
                        acc_sh.at[pl.ds(s * _RPT, _RPT)])

    @pl.when(c == 1)
    def _():
        pltpu.sync_copy(xf_hbm.at[pl.ds(s * _RPT, _RPT), pl.ds(_HD, _HD)],
                        x_sh.at[pl.ds(s * _RPT, _RPT)])
        pltpu.sync_copy(xf_hbm.at[pl.ds(s * _RPT, _RPT), pl.ds(_HD, _HD)],
                        acc_sh.at[pl.ds(s * _RPT, _RPT)])

    plsc.subcore_barrier()

    def fetch_idx(i, k):
        pltpu.async_copy(e_hbm.at[0, s, i], ibufs[k].at[0], isems[k])
        pltpu.async_copy(e_hbm.at[1, s, i], ibufs[k].at[1], isems[k])

    def wait_idx(i, k):
        pltpu.make_async_copy(e_hbm.at[0, s, i], ibufs[k].at[0],
                              isems[k]).wait()
        pltpu.make_async_copy(e_hbm.at[1, s, i], ibufs[k].at[1],
                              isems[k]).wait()

    for k in range(4):
        fetch_idx(k, k)
    for k in range(2):
        wait_idx(k, k)
        pltpu.async_copy(x_sh.at[ibufs[k].at[0]], rows[k], gsems[k])

    @pl.loop(0, _NCH, step=_NIB)
    def _(i0):
        for b in range(_NIB):
            i = i0 + b
            br = b % _NBUF
            bg = (b + 2) % _NBUF
            bi = (b + 4) % _NIB
            bgi = (b + 2) % _NIB

            @pl.when(i >= 2)
            def _():
                pltpu.make_async_copy(rows[bg],
                                      acc_sh.at[ibufs[(b - 2) % _NIB].at[1]],
                                      ssems[bg]).wait()

            @pl.when(i + 4 < _NCH)
            def _():
                fetch_idx(i + 4, bi)

            @pl.when(i + 2 < _NCH)
            def _():
                wait_idx(i + 2, bgi)
                pltpu.async_copy(x_sh.at[ibufs[bgi].at[0]], rows[bg],
                                 gsems[bg])

            pltpu.make_async_copy(x_sh.at[ibufs[b].at[0]], rows[br],
                                  gsems[br]).wait()
            pltpu.async_copy(rows[br], acc_sh.at[ibufs[b].at[1]], ssems[br],
                             add=True)

    pltpu.make_async_copy(rows[(_NCH - 2) % _NBUF],
                          acc_sh.at[ibufs[(_NCH - 2) % _NIB].at[1]],
                          ssems[(_NCH - 2) % _NBUF]).wait()
    pltpu.make_async_copy(rows[(_NCH - 1) % _NBUF],
                          acc_sh.at[ibufs[(_NCH - 1) % _NIB].at[1]],
                          ssems[(_NCH - 1) % _NBUF]).wait()

    plsc.subcore_barrier()
    pltpu.sync_copy(acc_sh.at[pl.ds(s * _RPT, _RPT)],
                    out_hbm.at[c, pl.ds(s * _RPT, _RPT)])


def _mlp_bn(a, b, w1_ref, b1_ref, w2_ref, b2_ref, g_ref, bt_ref):
    a = a.astype(jnp.float32)
    b = b.astype(jnp.float32)
    h = jnp.dot(a, w1_ref[:_HD], preferred_element_type=jnp.float32)
    h += jnp.dot(b, w1_ref[_HD:], preferred_element_type=jnp.float32)
    h = jnp.maximum(h + b1_ref[...], 0.0)
    h = jnp.dot(h, w2_ref[...], preferred_element_type=jnp.float32) + b2_ref[...]
    mu = jnp.mean(h, axis=0, keepdims=True)
    var = jnp.mean(jnp.square(h - mu), axis=0, keepdims=True)
    h = (h - mu) * lax.rsqrt(var + 1e-5) * g_ref[...] + bt_ref[...]
    return jnp.maximum(h, 0.0)


def _dense_body(p_ref, w1_ref, b1_ref, w2_ref, b2_ref,
                g_ref, bt_ref, o_ref):
    h = _mlp_bn(p_ref[0, :_N], p_ref[1, :_N],
                w1_ref, b1_ref, w2_ref, b2_ref, g_ref, bt_ref)
    o_ref[pl.ds(0, _N)] = h.astype(jnp.bfloat16)


_dense = pl.pallas_call(
    _dense_body,
    out_shape=jax.ShapeDtypeStruct((_NP, _D), jnp.bfloat16),
)


def _final_body(p_ref, batch_ref, w1_ref, b1_ref, w2_ref,
                b2_ref, g_ref, bt_ref, wh1_ref, bh1_ref, wh2_ref, bh2_ref,
                o_ref):
    h = _mlp_bn(p_ref[0, :_N], p_ref[1, :_N],
                w1_ref, b1_ref, w2_ref, b2_ref, g_ref, bt_ref)
    gids = lax.broadcasted_iota(jnp.int32, (_G, _N), 0)
    onehot = (batch_ref[...] == gids).astype(jnp.float32)
    pool = jnp.dot(onehot, h, preferred_element_type=jnp.float32)
    q = jnp.maximum(
        jnp.dot(pool, wh1_ref[...], preferred_element_type=jnp.float32)
        + bh1_ref[...], 0.0)
    o_ref[...] = jnp.dot(q, wh2_ref[...],
                         preferred_element_type=jnp.float32) + bh2_ref[...]


_final = pl.pallas_call(
    _final_body,
    out_shape=jax.ShapeDtypeStruct((_G, _OUT), jnp.float32),
)


def kernel(x, edge_index, batch, W1_0, b1_0, W2_0, b2_0, g_0, bt_0,
           W1_1, b1_1, W2_1, b2_1, g_1, bt_1,
           W1_2, b1_2, W2_2, b2_2, g_2, bt_2, Wh1, bh1, Wh2, bh2):
    e = jnp.pad(edge_index, ((0, 0), (0, _EPAD - _E)),
                constant_values=_NP - 1).reshape(2, _NS, _NCH, _CH)
    r2 = lambda v: v.reshape(1, -1)
    xp = jnp.concatenate([x, jnp.zeros((_NP - _N, _D), jnp.float32)])
    xp = xp.astype(jnp.bfloat16)

    p = _sc_agg(xp, e)
    h = _dense(p, W1_0, r2(b1_0), W2_0, r2(b2_0), r2(g_0), r2(bt_0))
    p = _sc_agg(h, e)
    h = _dense(p, W1_1, r2(b1_1), W2_1, r2(b2_1), r2(g_1), r2(bt_1))
    p = _sc_agg(h, e)
    return _final(p, batch.reshape(1, -1), W1_2, r2(b1_2), W2_2,
                  r2(b2_2), r2(g_2), r2(bt_2), Wh1, r2(bh1), Wh2, r2(bh2))
